# Initial kernel scaffold; baseline (speedup 1.0000x reference)
import jax
import jax.numpy as jnp
from jax.experimental import pallas as pl


def kernel(inputs, embedding):
    raise NotImplementedError("write your pallas kernel here")


# speedup vs baseline: 1.0301x; 1.0301x over previous
import jax
import jax.numpy as jnp
from jax.experimental import pallas as pl
from jax.experimental.pallas import tpu as pltpu
from jax.experimental.pallas import tpu_sc as plsc

CODEBOOK = 8192
DIM = 256
TOKENS = 8192
TOK_TILE = 256
CODE_CHUNK = 512
HALF = CODEBOOK // 2
CHUNKS_PER_HALF = HALF // CODE_CHUNK
N_TILES = TOKENS // TOK_TILE
GATHER_WINDOW = 128
COMMITMENT_COST = 0.25


def _argmin_body(x_ref, et_ref, idx_ref, e2_ref, etb_ref):
    i = pl.program_id(0)

    @pl.when(i == 0)
    def _():
        et = et_ref[...]
        e2_ref[...] = jnp.sum(et * et, axis=0, keepdims=True)
        etb_ref[...] = et.astype(jnp.bfloat16)

    x = x_ref[...]
    x2 = jnp.sum(x * x, axis=1, keepdims=True)
    xb = x.astype(jnp.bfloat16)

    half_min = []
    half_arg = []
    for h in range(2):
        best = jnp.full((TOK_TILE, 128), jnp.inf, jnp.float32)
        arg = jnp.zeros((TOK_TILE, 128), jnp.int32)
        for c in range(CHUNKS_PER_HALF):
            lo = h * HALF + c * CODE_CHUNK
            m = jax.lax.dot_general(
                xb,
                etb_ref[:, lo:lo + CODE_CHUNK],
                (((1,), (0,)), ((), ())),
                preferred_element_type=jnp.float32,
            )
            e2c = e2_ref[:, lo:lo + CODE_CHUNK]
            d = (x2 + e2c) - (m + m)
            for s in range(CODE_CHUNK // 128):
                dcol = d[:, s * 128:(s + 1) * 128]
                base = lo + s * 128
                mask = dcol < best
                best = jnp.where(mask, dcol, best)
                arg = jnp.where(mask, base, arg)

        full_idx = arg + jax.lax.broadcasted_iota(
            jnp.int32, (TOK_TILE, 128), 1)
        minv = jnp.min(best, axis=1, keepdims=True)
        cand = jnp.where(best == minv, full_idx, jnp.int32(2 ** 30))
        half_min.append(minv)
        half_arg.append(jnp.min(cand, axis=1, keepdims=True))

    carry = half_min[0].astype(jnp.bfloat16).astype(jnp.float32)
    take2 = half_min[1] < carry
    idx_ref[...] = jnp.where(take2, half_arg[1], half_arg[0])


def _argmin_indices(x, et):
    return pl.pallas_call(
        _argmin_body,
        grid=(N_TILES,),
        in_specs=[
            pl.BlockSpec((TOK_TILE, DIM), lambda i: (i, 0)),
            pl.BlockSpec((DIM, CODEBOOK), lambda i: (0, 0)),
        ],
        out_specs=pl.BlockSpec((TOK_TILE, 1), lambda i: (i, 0)),
        out_shape=jax.ShapeDtypeStruct((TOKENS, 1), jnp.int32),
        scratch_shapes=[
            pltpu.VMEM((1, CODEBOOK), jnp.float32),
            pltpu.VMEM((DIM, CODEBOOK), jnp.bfloat16),
        ],
    )(x, et)


def _sc_gather(embedding, idx_row):
    vector_mesh = plsc.VectorSubcoreMesh(
        core_axis_name="core", subcore_axis_name="subcore"
    )

    @pl.kernel(
        out_type=jax.ShapeDtypeStruct((TOKENS, DIM), jnp.float32),
        mesh=vector_mesh,
    )
    def gather_kernel(e_hbm, i_hbm, o_hbm):
        def body(i_vmem, o_vmem):
            pltpu.sync_copy(e_hbm.at[i_vmem.at[0]], o_vmem)

        pltpu.emit_pipeline(
            body,
            grid=(TOKENS // GATHER_WINDOW,),
            in_specs=[
                pl.BlockSpec((1, GATHER_WINDOW), index_map=lambda i: (0, i))
            ],
            out_specs=[
                pl.BlockSpec((GATHER_WINDOW, DIM), index_map=lambda i: (i, 0))
            ],
            core_axis_name=("core", "subcore"),
            dimension_semantics=(pltpu.PARALLEL,),
        )(i_hbm, o_hbm)

    return gather_kernel(embedding, idx_row)


def _sse_body(x_ref, q_ref, out_ref):
    i = pl.program_id(0)

    @pl.when(i == 0)
    def _():
        out_ref[...] = jnp.zeros((1, 1), jnp.float32)

    delta = q_ref[...] - x_ref[...]
    out_ref[...] += jnp.sum(delta * delta, axis=(0, 1), keepdims=True)


def _sse(x, q):
    return pl.pallas_call(
        _sse_body,
        grid=(N_TILES,),
        in_specs=[
            pl.BlockSpec((TOK_TILE, DIM), lambda i: (i, 0)),
            pl.BlockSpec((TOK_TILE, DIM), lambda i: (i, 0)),
        ],
        out_specs=pl.BlockSpec((1, 1), lambda i: (0, 0)),
        out_shape=jax.ShapeDtypeStruct((1, 1), jnp.float32),
    )(x, q)


def kernel(inputs, embedding):
    input_shape = inputs.shape
    x = inputs.reshape(-1, DIM)
    et = embedding.T

    idx = _argmin_indices(x, et).reshape(-1)

    q = _sc_gather(embedding, idx.reshape(1, TOKENS))

    sse = _sse(x, q)[0, 0]
    mse = sse / (TOKENS * DIM)
    commitment_loss = COMMITMENT_COST * mse
    q_latent_loss = mse
    vq_loss = q_latent_loss + commitment_loss

    quantized_st = q.reshape(input_shape)
    indices = idx.reshape(input_shape[:-1])
    return quantized_st, indices, vq_loss, commitment_loss, q_latent_loss
